# baseline (speedup 1.0000x reference)
import functools

import jax
import jax.numpy as jnp
from jax import lax
from jax.experimental import pallas as pl
from jax.experimental.pallas import tpu as pltpu
from jax.experimental.pallas import tpu_sc as plsc

NC, NS, L = 2, 16, 16
NW = NC * NS


def _make_sc_call(N, D, E, C):
    TW = N // NW
    n_chunks = TW // C

    mesh = plsc.VectorSubcoreMesh(core_axis_name="c", subcore_axis_name="s")

    @functools.partial(
        pl.kernel,
        out_type=jax.ShapeDtypeStruct((N, D), jnp.float32),
        mesh=mesh,
        scratch_types=[
            pltpu.VMEM((TW,), jnp.int32),
            pltpu.VMEM((L,), jnp.float32),
            pltpu.VMEM((C, D), jnp.float32),
            pltpu.VMEM((C, D), jnp.float32),
            pltpu.SemaphoreType.DMA,
        ],
    )
    def call(hs_hbm, expert_hbm, routes_hbm, prob_hbm, out_hbm,
             idx_v, prob_v, rows_v, hid_v, sem):
        wid = lax.axis_index("s") * NC + lax.axis_index("c")
        base = wid * TW
        pltpu.sync_copy(routes_hbm.at[pl.ds(base, TW)], idx_v)

        def chunk_body(k, carry):
            tok = base + k * C
            gather = pltpu.async_copy(
                expert_hbm.at[idx_v.at[pl.ds(k * C, C)]], rows_v, sem)
            pltpu.sync_copy(prob_hbm.at[pl.ds(tok, C)], prob_v)
            pltpu.sync_copy(hs_hbm.at[pl.ds(tok, C)], hid_v)
            gather.wait()

            pc = prob_v[...]
            dnums = lax.GatherDimensionNumbers(
                offset_dims=(), collapsed_slice_dims=(0,),
                start_index_map=(0,))
            ps = [
                lax.gather(pc, jnp.full((L, 1), t, jnp.int32), dnums,
                           slice_sizes=(1,),
                           mode=lax.GatherScatterMode.PROMISE_IN_BOUNDS)
                for t in range(C)
            ]

            def col_body(j, c2):
                sl = pl.ds(pl.multiple_of(j * L, L), L)
                for t in range(C):
                    plsc.addupdate(hid_v.at[t, sl], rows_v[t, sl] * ps[t])
                return c2

            lax.fori_loop(0, D // L, col_body, 0)
            pltpu.sync_copy(hid_v, out_hbm.at[pl.ds(tok, C)])
            return carry

        lax.fori_loop(0, n_chunks, chunk_body, 0)

    return call


def kernel(hidden_states, expert_output, routes, route_prob_max):
    b, s, d = hidden_states.shape
    e = expert_output.shape[0]
    n = b * s
    hs2 = hidden_states.reshape(n, d)
    routes_i32 = routes.astype(jnp.int32)
    out = _make_sc_call(n, d, e, C=16)(
        hs2, expert_output, routes_i32, route_prob_max)
    return out.reshape(b, s, d)
